# initial kernel scaffold (unmeasured)
import jax
import jax.numpy as jnp
from jax import lax
from jax.experimental import pallas as pl
from jax.experimental.pallas import tpu as pltpu


def kernel(
    u,
):
    def body(*refs):
        pass

    out_shape = jax.ShapeDtypeStruct(..., jnp.float32)
    return pl.pallas_call(body, out_shape=out_shape)(...)


# baseline (device time: 13117 ns/iter reference)
import jax
import jax.numpy as jnp
from jax import lax
from jax.experimental import pallas as pl
from jax.experimental.pallas import tpu as pltpu

NX_DEV, NY_DEV, NZ_DEV = 2, 2, 4


def kernel(u):
    nx, ny, nz = u.shape

    def body(
        u_ref,
        out_ref,
        sx_ref,
        sy_ref,
        sz_lo_ref,
        sz_hi_ref,
        hx_ref,
        hy_ref,
        hz_ref,
        send_sems,
        recv_sems,
    ):
        my_x = lax.axis_index("x")
        my_y = lax.axis_index("y")
        my_z = lax.axis_index("z")

        src_i = jnp.where(my_x == 0, nx - 1, 0)
        sx_ref[...] = u_ref[pl.ds(src_i, 1), :, :]
        src_j = jnp.where(my_y == 0, ny - 1, 0)
        sy_ref[...] = u_ref[:, pl.ds(src_j, 1), :]
        sz_hi_ref[...] = u_ref[:, :, pl.ds(nz - 1, 1)]
        sz_lo_ref[...] = u_ref[:, :, pl.ds(0, 1)]

        rdma_x = pltpu.make_async_remote_copy(
            src_ref=sx_ref,
            dst_ref=hx_ref.at[my_x],
            send_sem=send_sems.at[0],
            recv_sem=recv_sems.at[0],
            device_id=(1 - my_x, my_y, my_z),
            device_id_type=pl.DeviceIdType.MESH,
        )
        rdma_x.start()

        rdma_y = pltpu.make_async_remote_copy(
            src_ref=sy_ref,
            dst_ref=hy_ref.at[my_y],
            send_sem=send_sems.at[1],
            recv_sem=recv_sems.at[1],
            device_id=(my_x, 1 - my_y, my_z),
            device_id_type=pl.DeviceIdType.MESH,
        )
        rdma_y.start()

        @pl.when(my_z < NZ_DEV - 1)
        def _():
            pltpu.make_async_remote_copy(
                src_ref=sz_hi_ref,
                dst_ref=hz_ref.at[0],
                send_sem=send_sems.at[2],
                recv_sem=recv_sems.at[2],
                device_id=(my_x, my_y, my_z + 1),
                device_id_type=pl.DeviceIdType.MESH,
            ).start()

        @pl.when(my_z > 0)
        def _():
            pltpu.make_async_remote_copy(
                src_ref=sz_lo_ref,
                dst_ref=hz_ref.at[1],
                send_sem=send_sems.at[3],
                recv_sem=recv_sems.at[3],
                device_id=(my_x, my_y, my_z - 1),
                device_id_type=pl.DeviceIdType.MESH,
            ).start()

        rdma_x.wait()
        rdma_y.wait()

        @pl.when(my_z < NZ_DEV - 1)
        def _():
            pltpu.make_async_remote_copy(
                src_ref=sz_hi_ref,
                dst_ref=hz_ref.at[1],
                send_sem=send_sems.at[2],
                recv_sem=recv_sems.at[3],
                device_id=(my_x, my_y, my_z + 1),
                device_id_type=pl.DeviceIdType.MESH,
            ).wait()

        @pl.when(my_z > 0)
        def _():
            pltpu.make_async_remote_copy(
                src_ref=sz_lo_ref,
                dst_ref=hz_ref.at[0],
                send_sem=send_sems.at[3],
                recv_sem=recv_sems.at[2],
                device_id=(my_x, my_y, my_z - 1),
                device_id_type=pl.DeviceIdType.MESH,
            ).wait()

        uu = u_ref[...]
        u_xm = jnp.concatenate([hx_ref[0], uu[:-1, :, :]], axis=0)
        u_xp = jnp.concatenate([uu[1:, :, :], hx_ref[1]], axis=0)
        u_ym = jnp.concatenate([hy_ref[0], uu[:, :-1, :]], axis=1)
        u_yp = jnp.concatenate([uu[:, 1:, :], hy_ref[1]], axis=1)
        u_zm = jnp.concatenate([hz_ref[0], uu[:, :, :-1]], axis=2)
        u_zp = jnp.concatenate([uu[:, :, 1:], hz_ref[1]], axis=2)
        v = u_xm + u_xp + u_ym + u_yp + u_zm + u_zp - 6.0 * uu

        ii = lax.broadcasted_iota(jnp.int32, (nx, ny, nz), 0)
        jj = lax.broadcasted_iota(jnp.int32, (nx, ny, nz), 1)
        kk = lax.broadcasted_iota(jnp.int32, (nx, ny, nz), 2)
        boundary = (
            ((my_x == 0) & (ii == 0))
            | ((my_x == NX_DEV - 1) & (ii == nx - 1))
            | ((my_y == 0) & (jj == 0))
            | ((my_y == NY_DEV - 1) & (jj == ny - 1))
            | ((my_z == 0) & (kk == 0))
            | ((my_z == NZ_DEV - 1) & (kk == nz - 1))
        )
        out_ref[...] = jnp.where(boundary, 0.0, v)

    return pl.pallas_call(
        body,
        out_shape=jax.ShapeDtypeStruct((nx, ny, nz), jnp.float32),
        in_specs=[pl.BlockSpec(memory_space=pltpu.VMEM)],
        out_specs=pl.BlockSpec(memory_space=pltpu.VMEM),
        scratch_shapes=[
            pltpu.VMEM((1, ny, nz), jnp.float32),
            pltpu.VMEM((nx, 1, nz), jnp.float32),
            pltpu.VMEM((nx, ny, 1), jnp.float32),
            pltpu.VMEM((nx, ny, 1), jnp.float32),
            pltpu.VMEM((2, 1, ny, nz), jnp.float32),
            pltpu.VMEM((2, nx, 1, nz), jnp.float32),
            pltpu.VMEM((2, nx, ny, 1), jnp.float32),
            pltpu.SemaphoreType.DMA((4,)),
            pltpu.SemaphoreType.DMA((4,)),
        ],
    )(u)


# device time: 7968 ns/iter; 1.6462x vs baseline; 1.6462x over previous
import jax
import jax.numpy as jnp
from jax import lax
from jax.experimental import pallas as pl
from jax.experimental.pallas import tpu as pltpu

NX_DEV, NY_DEV, NZ_DEV = 2, 2, 4


def kernel(u):
    nx, ny, nz = u.shape

    def body(
        u_ref,
        out_ref,
        sx_ref,
        sy_ref,
        sz_lo_ref,
        sz_hi_ref,
        hx_ref,
        hy_ref,
        hz_ref,
        send_sems,
        recv_sems,
    ):
        my_x = lax.axis_index("x")
        my_y = lax.axis_index("y")
        my_z = lax.axis_index("z")
        has_below = my_z > 0
        has_above = my_z < NZ_DEV - 1

        barrier_sem = pltpu.get_barrier_semaphore()
        pl.semaphore_signal(
            barrier_sem, inc=1,
            device_id=(1 - my_x, my_y, my_z),
            device_id_type=pl.DeviceIdType.MESH,
        )
        pl.semaphore_signal(
            barrier_sem, inc=1,
            device_id=(my_x, 1 - my_y, my_z),
            device_id_type=pl.DeviceIdType.MESH,
        )

        @pl.when(has_above)
        def _():
            pl.semaphore_signal(
                barrier_sem, inc=1,
                device_id=(my_x, my_y, my_z + 1),
                device_id_type=pl.DeviceIdType.MESH,
            )

        @pl.when(has_below)
        def _():
            pl.semaphore_signal(
                barrier_sem, inc=1,
                device_id=(my_x, my_y, my_z - 1),
                device_id_type=pl.DeviceIdType.MESH,
            )

        n_neighbors = 2 + has_below.astype(jnp.int32) + has_above.astype(jnp.int32)
        pl.semaphore_wait(barrier_sem, n_neighbors)

        src_i = jnp.where(my_x == 0, nx - 1, 0)
        sx_ref[...] = u_ref[pl.ds(src_i, 1), :, :]
        src_j = jnp.where(my_y == 0, ny - 1, 0)
        sy_ref[...] = u_ref[:, pl.ds(src_j, 1), :]
        sz_hi_ref[...] = u_ref[:, :, pl.ds(nz - 1, 1)]
        sz_lo_ref[...] = u_ref[:, :, pl.ds(0, 1)]

        rdma_x = pltpu.make_async_remote_copy(
            src_ref=sx_ref,
            dst_ref=hx_ref.at[my_x],
            send_sem=send_sems.at[0],
            recv_sem=recv_sems.at[0],
            device_id=(1 - my_x, my_y, my_z),
            device_id_type=pl.DeviceIdType.MESH,
        )
        rdma_x.start()

        rdma_y = pltpu.make_async_remote_copy(
            src_ref=sy_ref,
            dst_ref=hy_ref.at[my_y],
            send_sem=send_sems.at[1],
            recv_sem=recv_sems.at[1],
            device_id=(my_x, 1 - my_y, my_z),
            device_id_type=pl.DeviceIdType.MESH,
        )
        rdma_y.start()

        @pl.when(has_above)
        def _():
            pltpu.make_async_remote_copy(
                src_ref=sz_hi_ref,
                dst_ref=hz_ref.at[0],
                send_sem=send_sems.at[2],
                recv_sem=recv_sems.at[2],
                device_id=(my_x, my_y, my_z + 1),
                device_id_type=pl.DeviceIdType.MESH,
            ).start()

        @pl.when(has_below)
        def _():
            pltpu.make_async_remote_copy(
                src_ref=sz_lo_ref,
                dst_ref=hz_ref.at[1],
                send_sem=send_sems.at[3],
                recv_sem=recv_sems.at[3],
                device_id=(my_x, my_y, my_z - 1),
                device_id_type=pl.DeviceIdType.MESH,
            ).start()

        uu = u_ref[...]
        zx = jnp.zeros((1, ny, nz), jnp.float32)
        zy = jnp.zeros((nx, 1, nz), jnp.float32)
        zz = jnp.zeros((nx, ny, 1), jnp.float32)
        v = (
            jnp.concatenate([zx, uu[:-1, :, :]], axis=0)
            + jnp.concatenate([uu[1:, :, :], zx], axis=0)
            + jnp.concatenate([zy, uu[:, :-1, :]], axis=1)
            + jnp.concatenate([uu[:, 1:, :], zy], axis=1)
            + jnp.concatenate([zz, uu[:, :, :-1]], axis=2)
            + jnp.concatenate([uu[:, :, 1:], zz], axis=2)
            - 6.0 * uu
        )

        ii = lax.broadcasted_iota(jnp.int32, (nx, ny, nz), 0)
        jj = lax.broadcasted_iota(jnp.int32, (nx, ny, nz), 1)
        kk = lax.broadcasted_iota(jnp.int32, (nx, ny, nz), 2)

        rdma_x.wait()
        rdma_y.wait()

        @pl.when(has_above)
        def _():
            pltpu.make_async_remote_copy(
                src_ref=sz_hi_ref,
                dst_ref=hz_ref.at[1],
                send_sem=send_sems.at[2],
                recv_sem=recv_sems.at[3],
                device_id=(my_x, my_y, my_z + 1),
                device_id_type=pl.DeviceIdType.MESH,
            ).wait()

        @pl.when(has_below)
        def _():
            pltpu.make_async_remote_copy(
                src_ref=sz_lo_ref,
                dst_ref=hz_ref.at[0],
                send_sem=send_sems.at[3],
                recv_sem=recv_sems.at[2],
                device_id=(my_x, my_y, my_z - 1),
                device_id_type=pl.DeviceIdType.MESH,
            ).wait()

        v = v + jnp.where((ii == 0) & (my_x > 0), hx_ref[0], 0.0)
        v = v + jnp.where((ii == nx - 1) & (my_x < NX_DEV - 1), hx_ref[1], 0.0)
        v = v + jnp.where((jj == 0) & (my_y > 0), hy_ref[0], 0.0)
        v = v + jnp.where((jj == ny - 1) & (my_y < NY_DEV - 1), hy_ref[1], 0.0)
        v = v + jnp.where((kk == 0) & has_below, hz_ref[0], 0.0)
        v = v + jnp.where((kk == nz - 1) & has_above, hz_ref[1], 0.0)

        boundary = (
            ((my_x == 0) & (ii == 0))
            | ((my_x == NX_DEV - 1) & (ii == nx - 1))
            | ((my_y == 0) & (jj == 0))
            | ((my_y == NY_DEV - 1) & (jj == ny - 1))
            | ((my_z == 0) & (kk == 0))
            | ((my_z == NZ_DEV - 1) & (kk == nz - 1))
        )
        out_ref[...] = jnp.where(boundary, 0.0, v)

    return pl.pallas_call(
        body,
        out_shape=jax.ShapeDtypeStruct((nx, ny, nz), jnp.float32),
        in_specs=[pl.BlockSpec(memory_space=pltpu.VMEM)],
        out_specs=pl.BlockSpec(memory_space=pltpu.VMEM),
        scratch_shapes=[
            pltpu.VMEM((1, ny, nz), jnp.float32),
            pltpu.VMEM((nx, 1, nz), jnp.float32),
            pltpu.VMEM((nx, ny, 1), jnp.float32),
            pltpu.VMEM((nx, ny, 1), jnp.float32),
            pltpu.VMEM((2, 1, ny, nz), jnp.float32),
            pltpu.VMEM((2, nx, 1, nz), jnp.float32),
            pltpu.VMEM((2, nx, ny, 1), jnp.float32),
            pltpu.SemaphoreType.DMA((4,)),
            pltpu.SemaphoreType.DMA((4,)),
        ],
        compiler_params=pltpu.CompilerParams(collective_id=0),
    )(u)


# device time: 7887 ns/iter; 1.6631x vs baseline; 1.0103x over previous
import jax
import jax.numpy as jnp
from jax import lax
from jax.experimental import pallas as pl
from jax.experimental.pallas import tpu as pltpu

NX_DEV, NY_DEV, NZ_DEV = 2, 2, 4


def kernel(u):
    nx, ny, nz = u.shape

    def body(
        u_ref,
        out_ref,
        sx_ref,
        sy_ref,
        sz_lo_ref,
        sz_hi_ref,
        hx_ref,
        hy_ref,
        hz_ref,
        send_sems,
        recv_sems,
    ):
        my_x = lax.axis_index("x")
        my_y = lax.axis_index("y")
        my_z = lax.axis_index("z")
        has_below = my_z > 0
        has_above = my_z < NZ_DEV - 1

        barrier_sem = pltpu.get_barrier_semaphore()
        pl.semaphore_signal(
            barrier_sem, inc=1,
            device_id=(1 - my_x, my_y, my_z),
            device_id_type=pl.DeviceIdType.MESH,
        )
        pl.semaphore_signal(
            barrier_sem, inc=1,
            device_id=(my_x, 1 - my_y, my_z),
            device_id_type=pl.DeviceIdType.MESH,
        )

        @pl.when(has_above)
        def _():
            pl.semaphore_signal(
                barrier_sem, inc=1,
                device_id=(my_x, my_y, my_z + 1),
                device_id_type=pl.DeviceIdType.MESH,
            )

        @pl.when(has_below)
        def _():
            pl.semaphore_signal(
                barrier_sem, inc=1,
                device_id=(my_x, my_y, my_z - 1),
                device_id_type=pl.DeviceIdType.MESH,
            )

        src_i = jnp.where(my_x == 0, nx - 1, 0)
        sx_ref[...] = u_ref[pl.ds(src_i, 1), :, :]
        src_j = jnp.where(my_y == 0, ny - 1, 0)
        sy_ref[...] = u_ref[:, pl.ds(src_j, 1), :]
        sz_hi_ref[...] = u_ref[:, :, pl.ds(nz - 1, 1)]
        sz_lo_ref[...] = u_ref[:, :, pl.ds(0, 1)]

        z_interior = has_below & has_above

        @pl.when(z_interior)
        def _():
            pl.semaphore_wait(barrier_sem, 4)

        @pl.when(jnp.logical_not(z_interior))
        def _():
            pl.semaphore_wait(barrier_sem, 3)

        rdma_x = pltpu.make_async_remote_copy(
            src_ref=sx_ref,
            dst_ref=hx_ref.at[my_x],
            send_sem=send_sems.at[0],
            recv_sem=recv_sems.at[0],
            device_id=(1 - my_x, my_y, my_z),
            device_id_type=pl.DeviceIdType.MESH,
        )
        rdma_x.start()

        rdma_y = pltpu.make_async_remote_copy(
            src_ref=sy_ref,
            dst_ref=hy_ref.at[my_y],
            send_sem=send_sems.at[1],
            recv_sem=recv_sems.at[1],
            device_id=(my_x, 1 - my_y, my_z),
            device_id_type=pl.DeviceIdType.MESH,
        )
        rdma_y.start()

        @pl.when(has_above)
        def _():
            pltpu.make_async_remote_copy(
                src_ref=sz_hi_ref,
                dst_ref=hz_ref.at[0],
                send_sem=send_sems.at[2],
                recv_sem=recv_sems.at[2],
                device_id=(my_x, my_y, my_z + 1),
                device_id_type=pl.DeviceIdType.MESH,
            ).start()

        @pl.when(has_below)
        def _():
            pltpu.make_async_remote_copy(
                src_ref=sz_lo_ref,
                dst_ref=hz_ref.at[1],
                send_sem=send_sems.at[3],
                recv_sem=recv_sems.at[3],
                device_id=(my_x, my_y, my_z - 1),
                device_id_type=pl.DeviceIdType.MESH,
            ).start()

        uu = u_ref[...]
        zx = jnp.zeros((1, ny, nz), jnp.float32)
        zy = jnp.zeros((nx, 1, nz), jnp.float32)
        zz = jnp.zeros((nx, ny, 1), jnp.float32)
        v = (
            jnp.concatenate([zx, uu[:-1, :, :]], axis=0)
            + jnp.concatenate([uu[1:, :, :], zx], axis=0)
            + jnp.concatenate([zy, uu[:, :-1, :]], axis=1)
            + jnp.concatenate([uu[:, 1:, :], zy], axis=1)
            + jnp.concatenate([zz, uu[:, :, :-1]], axis=2)
            + jnp.concatenate([uu[:, :, 1:], zz], axis=2)
            - 6.0 * uu
        )

        ii = lax.broadcasted_iota(jnp.int32, (nx, ny, nz), 0)
        jj = lax.broadcasted_iota(jnp.int32, (nx, ny, nz), 1)
        kk = lax.broadcasted_iota(jnp.int32, (nx, ny, nz), 2)

        rdma_x.wait_recv()
        rdma_y.wait_recv()

        @pl.when(has_below)
        def _():
            pltpu.make_async_remote_copy(
                src_ref=sz_lo_ref,
                dst_ref=hz_ref.at[0],
                send_sem=send_sems.at[3],
                recv_sem=recv_sems.at[2],
                device_id=(my_x, my_y, my_z),
                device_id_type=pl.DeviceIdType.MESH,
            ).wait_recv()

        @pl.when(has_above)
        def _():
            pltpu.make_async_remote_copy(
                src_ref=sz_hi_ref,
                dst_ref=hz_ref.at[1],
                send_sem=send_sems.at[2],
                recv_sem=recv_sems.at[3],
                device_id=(my_x, my_y, my_z),
                device_id_type=pl.DeviceIdType.MESH,
            ).wait_recv()

        v = v + jnp.where((ii == 0) & (my_x > 0), hx_ref[0], 0.0)
        v = v + jnp.where((ii == nx - 1) & (my_x < NX_DEV - 1), hx_ref[1], 0.0)
        v = v + jnp.where((jj == 0) & (my_y > 0), hy_ref[0], 0.0)
        v = v + jnp.where((jj == ny - 1) & (my_y < NY_DEV - 1), hy_ref[1], 0.0)
        v = v + jnp.where((kk == 0) & has_below, hz_ref[0], 0.0)
        v = v + jnp.where((kk == nz - 1) & has_above, hz_ref[1], 0.0)

        boundary = (
            ((my_x == 0) & (ii == 0))
            | ((my_x == NX_DEV - 1) & (ii == nx - 1))
            | ((my_y == 0) & (jj == 0))
            | ((my_y == NY_DEV - 1) & (jj == ny - 1))
            | ((my_z == 0) & (kk == 0))
            | ((my_z == NZ_DEV - 1) & (kk == nz - 1))
        )
        out_ref[...] = jnp.where(boundary, 0.0, v)

        rdma_x.wait_send()
        rdma_y.wait_send()

        @pl.when(has_above)
        def _():
            pltpu.make_async_remote_copy(
                src_ref=sz_hi_ref,
                dst_ref=hz_ref.at[0],
                send_sem=send_sems.at[2],
                recv_sem=recv_sems.at[2],
                device_id=(my_x, my_y, my_z),
                device_id_type=pl.DeviceIdType.MESH,
            ).wait_send()

        @pl.when(has_below)
        def _():
            pltpu.make_async_remote_copy(
                src_ref=sz_lo_ref,
                dst_ref=hz_ref.at[1],
                send_sem=send_sems.at[3],
                recv_sem=recv_sems.at[3],
                device_id=(my_x, my_y, my_z),
                device_id_type=pl.DeviceIdType.MESH,
            ).wait_send()

    return pl.pallas_call(
        body,
        out_shape=jax.ShapeDtypeStruct((nx, ny, nz), jnp.float32),
        in_specs=[pl.BlockSpec(memory_space=pltpu.VMEM)],
        out_specs=pl.BlockSpec(memory_space=pltpu.VMEM),
        scratch_shapes=[
            pltpu.VMEM((1, ny, nz), jnp.float32),
            pltpu.VMEM((nx, 1, nz), jnp.float32),
            pltpu.VMEM((nx, ny, 1), jnp.float32),
            pltpu.VMEM((nx, ny, 1), jnp.float32),
            pltpu.VMEM((2, 1, ny, nz), jnp.float32),
            pltpu.VMEM((2, nx, 1, nz), jnp.float32),
            pltpu.VMEM((2, nx, ny, 1), jnp.float32),
            pltpu.SemaphoreType.DMA((4,)),
            pltpu.SemaphoreType.DMA((4,)),
        ],
        compiler_params=pltpu.CompilerParams(collective_id=0),
    )(u)
